# baseline (speedup 1.0000x reference)
import functools

import jax
import jax.numpy as jnp
from jax import lax
from jax.experimental import pallas as pl
from jax.experimental.pallas import tpu as pltpu
from jax.experimental.pallas import tpu_sc as plsc

_NC = 2
_NS = 16
_NW = _NC * _NS
_CHUNK = 128


def _sc_gather(uids, iids, user_emb, item_emb):
    b = uids.shape[0]
    d = user_emb.shape[1]
    rows_per_w = b // _NW
    nchunk = rows_per_w // _CHUNK
    nslot = min(7, 2 * nchunk)
    mesh = plsc.VectorSubcoreMesh(core_axis_name="c", subcore_axis_name="s")

    @functools.partial(
        pl.kernel,
        out_type=(
            jax.ShapeDtypeStruct((b, d), jnp.float32),
            jax.ShapeDtypeStruct((b, d), jnp.float32),
        ),
        mesh=mesh,
        scratch_types=[
            pltpu.VMEM((rows_per_w,), jnp.int32),
            pltpu.VMEM((rows_per_w,), jnp.int32),
            pltpu.VMEM((nslot * _CHUNK, d), jnp.float32),
            pltpu.SemaphoreType.DMA((nslot,)),
            pltpu.SemaphoreType.DMA((nslot,)),
        ],
    )
    def gather_kernel(u_hbm, i_hbm, ue_hbm, ie_hbm, out_u, out_i,
                      uidx_v, iidx_v, rows_v, sem_g, sem_o):
        wid = lax.axis_index("s") * _NC + lax.axis_index("c")
        base = wid * rows_per_w
        pltpu.sync_copy(u_hbm.at[pl.ds(base, rows_per_w)], uidx_v)
        pltpu.sync_copy(i_hbm.at[pl.ds(base, rows_per_w)], iidx_v)
        work = []
        for j in range(nchunk):
            work.append((uidx_v, ue_hbm, out_u, j * _CHUNK))
            work.append((iidx_v, ie_hbm, out_i, j * _CHUNK))
        nwork = len(work)

        def gather(k):
            idx_v, table_hbm, _, src_off = work[k]
            slot = k % nslot
            return pltpu.async_copy(
                table_hbm.at[idx_v.at[pl.ds(src_off, _CHUNK)]],
                rows_v.at[pl.ds(slot * _CHUNK, _CHUNK)],
                sem_g.at[slot],
            )

        gh = [gather(k) for k in range(min(nslot, nwork))]
        gh += [None] * (nwork - len(gh))
        outs = [None] * nslot
        for k in range(nwork):
            slot = k % nslot
            if gh[k] is None:
                outs[slot].wait()
                gh[k] = gather(k)
            gh[k].wait()
            _, _, out_hbm, src_off = work[k]
            outs[slot] = pltpu.async_copy(
                rows_v.at[pl.ds(slot * _CHUNK, _CHUNK)],
                out_hbm.at[pl.ds(base + src_off, _CHUNK)],
                sem_o.at[slot],
            )
        for o in outs:
            if o is not None:
                o.wait()

    return gather_kernel(uids, iids, user_emb, item_emb)


def _mlp_body(u_ref, i_ref, w1_ref, b1_ref, w2_ref, b2_ref,
              w3_ref, b3_ref, o_ref):
    d = u_ref.shape[1]
    ct = (((1,), (1,)), ((), ()))
    h = lax.dot_general(u_ref[...], w1_ref[:, :d], ct,
                        preferred_element_type=jnp.float32)
    h = h + lax.dot_general(i_ref[...], w1_ref[:, d:], ct,
                            preferred_element_type=jnp.float32)
    h = jnp.maximum(h + b1_ref[...], 0.0)
    h2 = lax.dot_general(h, w2_ref[...], ct,
                         preferred_element_type=jnp.float32)
    h2 = jnp.maximum(h2 + b2_ref[...], 0.0)
    z = lax.dot_general(w3_ref[...], h2, ct,
                        preferred_element_type=jnp.float32)
    o_ref[...] = z.reshape(o_ref.shape) + b3_ref[0]


def _tc_mlp(ug, ig, W1, b1, W2, b2, W3, b3, interpret=False):
    b, d = ug.shape
    tile = 8192
    b1r = b1.reshape(1, -1)
    b2r = b2.reshape(1, -1)
    grid = (b // tile,)
    full = lambda shape: pl.BlockSpec(shape, lambda i: (0,) * len(shape))
    return pl.pallas_call(
        _mlp_body,
        grid=grid,
        in_specs=[
            pl.BlockSpec((tile, d), lambda i: (i, 0)),
            pl.BlockSpec((tile, d), lambda i: (i, 0)),
            full(W1.shape),
            full(b1r.shape),
            full(W2.shape),
            full(b2r.shape),
            full(W3.shape),
            pl.BlockSpec(memory_space=pltpu.SMEM),
        ],
        out_specs=pl.BlockSpec((tile,), lambda i: (i,)),
        out_shape=jax.ShapeDtypeStruct((b,), jnp.float32),
        interpret=interpret,
    )(ug, ig, W1, b1r, W2, b2r, W3, b3)


def kernel(user_ids, item_ids, user_emb, item_emb, W1, b1, W2, b2, W3, b3):
    uids = user_ids.astype(jnp.int32)
    iids = item_ids.astype(jnp.int32)
    ug, ig = _sc_gather(uids, iids, user_emb, item_emb)
    return _tc_mlp(ug, ig, W1, b1, W2, b2, W3, b3)


# speedup vs baseline: 1.0099x; 1.0099x over previous
import functools

import jax
import jax.numpy as jnp
from jax import lax
from jax.experimental import pallas as pl
from jax.experimental.pallas import tpu as pltpu
from jax.experimental.pallas import tpu_sc as plsc

_NC = 2
_NS = 16
_NW = _NC * _NS
_CHUNK = 128


def _sc_gather(uids, iids, user_emb, item_emb):
    b = uids.shape[0]
    d = user_emb.shape[1]
    rows_per_w = b // _NW
    nchunk = rows_per_w // _CHUNK
    nslot = min(7, 2 * nchunk)
    mesh = plsc.VectorSubcoreMesh(core_axis_name="c", subcore_axis_name="s")

    @functools.partial(
        pl.kernel,
        out_type=jax.ShapeDtypeStruct((b, 2 * d), jnp.float32),
        mesh=mesh,
        scratch_types=[
            pltpu.VMEM((rows_per_w,), jnp.int32),
            pltpu.VMEM((rows_per_w,), jnp.int32),
            pltpu.VMEM((nslot * _CHUNK, d), jnp.float32),
            pltpu.SemaphoreType.DMA((nslot,)),
            pltpu.SemaphoreType.DMA((nslot,)),
        ],
    )
    def gather_kernel(u_hbm, i_hbm, ue_hbm, ie_hbm, out,
                      uidx_v, iidx_v, rows_v, sem_g, sem_o):
        wid = lax.axis_index("s") * _NC + lax.axis_index("c")
        base = wid * rows_per_w
        pltpu.sync_copy(u_hbm.at[pl.ds(base, rows_per_w)], uidx_v)
        pltpu.sync_copy(i_hbm.at[pl.ds(base, rows_per_w)], iidx_v)
        work = []
        for j in range(nchunk):
            work.append((uidx_v, ue_hbm, 0, j * _CHUNK))
            work.append((iidx_v, ie_hbm, d, j * _CHUNK))
        nwork = len(work)

        def gather(k):
            idx_v, table_hbm, _, src_off = work[k]
            slot = k % nslot
            return pltpu.async_copy(
                table_hbm.at[idx_v.at[pl.ds(src_off, _CHUNK)]],
                rows_v.at[pl.ds(slot * _CHUNK, _CHUNK)],
                sem_g.at[slot],
            )

        gh = [gather(k) for k in range(min(nslot, nwork))]
        gh += [None] * (nwork - len(gh))
        outs = [None] * nslot
        for k in range(nwork):
            slot = k % nslot
            if gh[k] is None:
                outs[slot].wait()
                gh[k] = gather(k)
            gh[k].wait()
            _, _, col_off, src_off = work[k]
            outs[slot] = pltpu.async_copy(
                rows_v.at[pl.ds(slot * _CHUNK, _CHUNK)],
                out.at[pl.ds(base + src_off, _CHUNK), pl.ds(col_off, d)],
                sem_o.at[slot],
            )
        for o in outs:
            if o is not None:
                o.wait()

    return gather_kernel(uids, iids, user_emb, item_emb)


def _mlp_body(x_ref, w1_ref, b1_ref, w2_ref, b2_ref, w3_ref, b3_ref, o_ref):
    ct = (((1,), (1,)), ((), ()))
    h = lax.dot_general(x_ref[...], w1_ref[...], ct,
                        preferred_element_type=jnp.float32)
    h = jnp.maximum(h + b1_ref[...], 0.0)
    h2 = lax.dot_general(h, w2_ref[...], ct,
                         preferred_element_type=jnp.float32)
    h2 = jnp.maximum(h2 + b2_ref[...], 0.0)
    z = lax.dot_general(w3_ref[...], h2, ct,
                        preferred_element_type=jnp.float32)
    o_ref[...] = z.reshape(o_ref.shape) + b3_ref[0]


def _tc_mlp(x, W1, b1, W2, b2, W3, b3, interpret=False):
    b, d2 = x.shape
    tile = 4096
    b1r = b1.reshape(1, -1)
    b2r = b2.reshape(1, -1)
    grid = (b // tile,)
    full = lambda shape: pl.BlockSpec(shape, lambda i: (0,) * len(shape))
    return pl.pallas_call(
        _mlp_body,
        grid=grid,
        in_specs=[
            pl.BlockSpec((tile, d2), lambda i: (i, 0)),
            full(W1.shape),
            full(b1r.shape),
            full(W2.shape),
            full(b2r.shape),
            full(W3.shape),
            pl.BlockSpec(memory_space=pltpu.SMEM),
        ],
        out_specs=pl.BlockSpec((tile,), lambda i: (i,)),
        out_shape=jax.ShapeDtypeStruct((b,), jnp.float32),
        interpret=interpret,
    )(x, W1, b1r, W2, b2r, W3, b3)


def kernel(user_ids, item_ids, user_emb, item_emb, W1, b1, W2, b2, W3, b3):
    uids = user_ids.astype(jnp.int32)
    iids = item_ids.astype(jnp.int32)
    x = _sc_gather(uids, iids, user_emb, item_emb)
    return _tc_mlp(x, W1, b1, W2, b2, W3, b3)


# speedup vs baseline: 1.0168x; 1.0068x over previous
import functools

import jax
import jax.numpy as jnp
from jax import lax
from jax.experimental import pallas as pl
from jax.experimental.pallas import tpu as pltpu
from jax.experimental.pallas import tpu_sc as plsc

_NC = 2
_NS = 16
_NW = _NC * _NS
_CHUNK = 128


def _sc_gather(uids, iids, user_emb, item_emb):
    b = uids.shape[0]
    d = user_emb.shape[1]
    rows_per_w = b // _NW
    nchunk = rows_per_w // _CHUNK
    nslot = min(7, 2 * nchunk)
    mesh = plsc.VectorSubcoreMesh(core_axis_name="c", subcore_axis_name="s")

    @functools.partial(
        pl.kernel,
        out_type=jax.ShapeDtypeStruct((b, 2 * d), jnp.float32),
        mesh=mesh,
        scratch_types=[
            pltpu.VMEM((rows_per_w,), jnp.int32),
            pltpu.VMEM((rows_per_w,), jnp.int32),
            pltpu.VMEM((nslot * _CHUNK, d), jnp.float32),
            pltpu.SemaphoreType.DMA((nslot,)),
            pltpu.SemaphoreType.DMA((nslot,)),
            pltpu.SemaphoreType.DMA((2,)),
        ],
    )
    def gather_kernel(u_hbm, i_hbm, ue_hbm, ie_hbm, out,
                      uidx_v, iidx_v, rows_v, sem_g, sem_o, sem_idx):
        wid = lax.axis_index("s") * _NC + lax.axis_index("c")
        base = wid * rows_per_w
        h_uidx = pltpu.async_copy(u_hbm.at[pl.ds(base, rows_per_w)], uidx_v,
                                  sem_idx.at[0])
        h_iidx = pltpu.async_copy(i_hbm.at[pl.ds(base, rows_per_w)], iidx_v,
                                  sem_idx.at[1])
        work = []
        for j in range(nchunk):
            work.append((uidx_v, ue_hbm, 0, j * _CHUNK))
        for j in range(nchunk):
            work.append((iidx_v, ie_hbm, d, j * _CHUNK))
        nwork = len(work)
        h_uidx.wait()

        def gather(k):
            if k == nchunk:
                h_iidx.wait()
            idx_v, table_hbm, _, src_off = work[k]
            slot = k % nslot
            return pltpu.async_copy(
                table_hbm.at[idx_v.at[pl.ds(src_off, _CHUNK)]],
                rows_v.at[pl.ds(slot * _CHUNK, _CHUNK)],
                sem_g.at[slot],
            )

        gh = [gather(k) for k in range(min(nslot, nwork))]
        gh += [None] * (nwork - len(gh))
        outs = [None] * nslot
        for k in range(nwork):
            slot = k % nslot
            if gh[k] is None:
                outs[slot].wait()
                gh[k] = gather(k)
            gh[k].wait()
            _, _, col_off, src_off = work[k]
            outs[slot] = pltpu.async_copy(
                rows_v.at[pl.ds(slot * _CHUNK, _CHUNK)],
                out.at[pl.ds(base + src_off, _CHUNK), pl.ds(col_off, d)],
                sem_o.at[slot],
            )
        for o in outs:
            if o is not None:
                o.wait()

    return gather_kernel(uids, iids, user_emb, item_emb)


def _mlp_body(x_ref, w1_ref, b1_ref, w2_ref, b2_ref, w3_ref, b3_ref, o_ref):
    ct = (((1,), (1,)), ((), ()))
    h = lax.dot_general(x_ref[...], w1_ref[...], ct,
                        preferred_element_type=jnp.float32)
    h = jnp.maximum(h + b1_ref[...], 0.0)
    h2 = lax.dot_general(h, w2_ref[...], ct,
                         preferred_element_type=jnp.float32)
    h2 = jnp.maximum(h2 + b2_ref[...], 0.0)
    z = lax.dot_general(w3_ref[...], h2, ct,
                        preferred_element_type=jnp.float32)
    o_ref[...] = z.reshape(o_ref.shape) + b3_ref[0]


def _tc_mlp(x, W1, b1, W2, b2, W3, b3, interpret=False):
    b, d2 = x.shape
    tile = 4096
    b1r = b1.reshape(1, -1)
    b2r = b2.reshape(1, -1)
    grid = (b // tile,)
    full = lambda shape: pl.BlockSpec(shape, lambda i: (0,) * len(shape))
    return pl.pallas_call(
        _mlp_body,
        grid=grid,
        in_specs=[
            pl.BlockSpec((tile, d2), lambda i: (i, 0)),
            full(W1.shape),
            full(b1r.shape),
            full(W2.shape),
            full(b2r.shape),
            full(W3.shape),
            pl.BlockSpec(memory_space=pltpu.SMEM),
        ],
        out_specs=pl.BlockSpec((tile,), lambda i: (i,)),
        out_shape=jax.ShapeDtypeStruct((b,), jnp.float32),
        interpret=interpret,
    )(x, W1, b1r, W2, b2r, W3, b3)


def kernel(user_ids, item_ids, user_emb, item_emb, W1, b1, W2, b2, W3, b3):
    uids = user_ids.astype(jnp.int32)
    iids = item_ids.astype(jnp.int32)
    x = _sc_gather(uids, iids, user_emb, item_emb)
    return _tc_mlp(x, W1, b1, W2, b2, W3, b3)


# speedup vs baseline: 1.0323x; 1.0152x over previous
import functools

import jax
import jax.numpy as jnp
import numpy as np
from jax import lax
from jax.experimental import pallas as pl
from jax.experimental.pallas import tpu as pltpu
from jax.experimental.pallas import tpu_sc as plsc

_NC = 2
_NS = 16
_NW = _NC * _NS
_CHUNK = 128
_LANES = 16


def _half_perms(d):
    j = np.arange(d // 2)
    c, t = j // _LANES, j % _LANES
    return 2 * _LANES * c + t, 2 * _LANES * c + _LANES + t


def _sc_gather(uids, iids, user_emb, item_emb):
    b = uids.shape[0]
    d = user_emb.shape[1]
    rows_per_w = b // _NW
    nchunk = rows_per_w // _CHUNK
    nslot = min(2, nchunk)
    nword = d // 2
    mesh = plsc.VectorSubcoreMesh(core_axis_name="c", subcore_axis_name="s")

    @functools.partial(
        pl.kernel,
        out_type=jax.ShapeDtypeStruct((b, d), jnp.int32),
        mesh=mesh,
        scratch_types=[
            pltpu.VMEM((rows_per_w,), jnp.int32),
            pltpu.VMEM((rows_per_w,), jnp.int32),
            pltpu.VMEM((2 * nslot * _CHUNK, d), jnp.float32),
            pltpu.VMEM((nslot * _CHUNK, d), jnp.int32),
            pltpu.SemaphoreType.DMA((2 * nslot,)),
            pltpu.SemaphoreType.DMA((nslot,)),
            pltpu.SemaphoreType.DMA((2,)),
        ],
    )
    def gather_kernel(u_hbm, i_hbm, ue_hbm, ie_hbm, out,
                      uidx_v, iidx_v, rows_v, pk_v, sem_g, sem_o, sem_idx):
        wid = lax.axis_index("s") * _NC + lax.axis_index("c")
        base = wid * rows_per_w
        h_uidx = pltpu.async_copy(u_hbm.at[pl.ds(base, rows_per_w)], uidx_v,
                                  sem_idx.at[0])
        h_iidx = pltpu.async_copy(i_hbm.at[pl.ds(base, rows_per_w)], iidx_v,
                                  sem_idx.at[1])
        h_uidx.wait()
        h_iidx.wait()

        def gather(j):
            slot = j % nslot
            hs = []
            for half, (idx_v, table_hbm) in enumerate(
                    ((uidx_v, ue_hbm), (iidx_v, ie_hbm))):
                buf = 2 * slot + half
                hs.append(pltpu.async_copy(
                    table_hbm.at[idx_v.at[pl.ds(j * _CHUNK, _CHUNK)]],
                    rows_v.at[pl.ds(buf * _CHUNK, _CHUNK)],
                    sem_g.at[buf],
                ))
            return hs

        half_bit = jnp.int32(0x8000)
        hi_mask = jnp.int32(-65536)

        def convert(slot):
            def row_body(r, carry):
                row = slot * _CHUNK + r
                for half in range(2):
                    src = (2 * slot + half) * _CHUNK + r
                    for c in range(d // (2 * _LANES)):
                        a = lax.bitcast_convert_type(
                            rows_v[src, pl.ds(2 * _LANES * c, _LANES)],
                            jnp.int32)
                        bb = lax.bitcast_convert_type(
                            rows_v[src,
                                   pl.ds(2 * _LANES * c + _LANES, _LANES)],
                            jnp.int32)
                        lo = lax.shift_right_logical(a + half_bit, 16)
                        hi = (bb + half_bit) & hi_mask
                        pk_v[row, pl.ds(half * nword + _LANES * c, _LANES)
                             ] = lo | hi
                return carry
            lax.fori_loop(0, _CHUNK, row_body, 0)

        gh = [gather(j) for j in range(min(nslot, nchunk))]
        gh += [None] * (nchunk - len(gh))
        outs = [None] * nslot
        for j in range(nchunk):
            slot = j % nslot
            for h in gh[j]:
                h.wait()
            if outs[slot] is not None:
                outs[slot].wait()
            convert(slot)
            outs[slot] = pltpu.async_copy(
                pk_v.at[pl.ds(slot * _CHUNK, _CHUNK)],
                out.at[pl.ds(base + j * _CHUNK, _CHUNK)],
                sem_o.at[slot],
            )
            nj = j + nslot
            if nj < nchunk:
                gh[nj] = gather(nj)
        for o in outs:
            if o is not None:
                o.wait()

    return gather_kernel(uids, iids, user_emb, item_emb)


def _mlp_body(x_ref, w1lo_ref, w1hi_ref, b1_ref, w2_ref, b2_ref,
              w3_ref, b3_ref, o_ref):
    ct = (((1,), (1,)), ((), ()))
    x = x_ref[...]
    xlo = lax.bitcast_convert_type(lax.shift_left(x, 16), jnp.float32)
    xhi = lax.bitcast_convert_type(x & jnp.int32(-65536), jnp.float32)
    h = lax.dot_general(xlo, w1lo_ref[...], ct,
                        preferred_element_type=jnp.float32)
    h = h + lax.dot_general(xhi, w1hi_ref[...], ct,
                            preferred_element_type=jnp.float32)
    h = jnp.maximum(h + b1_ref[...], 0.0)
    h2 = lax.dot_general(h, w2_ref[...], ct,
                         preferred_element_type=jnp.float32)
    h2 = jnp.maximum(h2 + b2_ref[...], 0.0)
    z = lax.dot_general(w3_ref[...], h2, ct,
                        preferred_element_type=jnp.float32)
    o_ref[...] = z.reshape(o_ref.shape) + b3_ref[0]


def _tc_mlp(x, W1lo, W1hi, b1, W2, b2, W3, b3, interpret=False):
    b, d = x.shape
    tile = 4096
    b1r = b1.reshape(1, -1)
    b2r = b2.reshape(1, -1)
    grid = (b // tile,)
    full = lambda shape: pl.BlockSpec(shape, lambda i: (0,) * len(shape))
    return pl.pallas_call(
        _mlp_body,
        grid=grid,
        in_specs=[
            pl.BlockSpec((tile, d), lambda i: (i, 0)),
            full(W1lo.shape),
            full(W1hi.shape),
            full(b1r.shape),
            full(W2.shape),
            full(b2r.shape),
            full(W3.shape),
            pl.BlockSpec(memory_space=pltpu.SMEM),
        ],
        out_specs=pl.BlockSpec((tile,), lambda i: (i,)),
        out_shape=jax.ShapeDtypeStruct((b,), jnp.float32),
        interpret=interpret,
    )(x, W1lo, W1hi, b1r, W2, b2r, W3, b3)


def kernel(user_ids, item_ids, user_emb, item_emb, W1, b1, W2, b2, W3, b3):
    uids = user_ids.astype(jnp.int32)
    iids = item_ids.astype(jnp.int32)
    d = user_emb.shape[1]
    x = _sc_gather(uids, iids, user_emb, item_emb)
    lo_u, hi_u = _half_perms(2 * d)
    lo = np.concatenate([lo_u[: d // 2], d + lo_u[: d // 2]])
    hi = np.concatenate([hi_u[: d // 2], d + hi_u[: d // 2]])
    W1lo = W1[:, lo]
    W1hi = W1[:, hi]
    return _tc_mlp(x, W1lo, W1hi, b1, W2, b2, W3, b3)
